# baseline (speedup 1.0000x reference)
import functools

import jax
import jax.numpy as jnp
from jax import lax
from jax.experimental import pallas as pl
from jax.experimental.pallas import tpu as pltpu
from jax.experimental.pallas import tpu_sc as plsc

NUM_EXPERTS = 8
TOP_K = 2
HIDDEN = 1024
INTER = 2048
TOKENS = 2048

TILE = 128
PAD = 5120
NT = PAD // TILE
NPAIR = TOKENS * TOP_K

NC = 2
NS = 16
NW = NC * NS
LANES = 16

_mesh = plsc.VectorSubcoreMesh(core_axis_name="c", subcore_axis_name="s")


def _wid():
    return lax.axis_index("s") * NC + lax.axis_index("c")


def _route_sort_body(h_ref, r_ref, pos_ref, w_ref, te_ref):
    h = h_ref[...]
    r = r_ref[...]
    logits = lax.dot_general(
        h, r, (((1,), (1,)), ((), ())),
        preferred_element_type=jnp.float32,
    )
    iota8 = lax.broadcasted_iota(jnp.int32, logits.shape, 1)
    m1 = jnp.max(logits, axis=1, keepdims=True)
    i1 = jnp.min(jnp.where(logits == m1, iota8, NUM_EXPERTS), axis=1,
                 keepdims=True)
    masked = jnp.where(iota8 == i1, -jnp.inf, logits)
    m2 = jnp.max(masked, axis=1, keepdims=True)
    i2 = jnp.min(jnp.where(masked == m2, iota8, NUM_EXPERTS), axis=1,
                 keepdims=True)
    w1 = 1.0 / (1.0 + jnp.exp(m2 - m1))
    w2 = 1.0 - w1

    one1 = (iota8 == i1).astype(jnp.float32)
    one2 = (iota8 == i2).astype(jnp.float32)
    occ = one1 + one2

    rr = lax.broadcasted_iota(jnp.int32, (TOKENS, TOKENS), 0)
    cc = lax.broadcasted_iota(jnp.int32, (TOKENS, TOKENS), 1)
    ltri = (cc < rr).astype(jnp.float32)
    csum = lax.dot_general(
        ltri, occ, (((1,), (0,)), ((), ())),
        preferred_element_type=jnp.float32,
    )

    n_tot = jnp.sum(occ, axis=0, keepdims=True)
    pe = jnp.floor((n_tot + (TILE - 1)) * (1.0 / TILE)) * TILE
    er = lax.broadcasted_iota(jnp.int32, (NUM_EXPERTS, NUM_EXPERTS), 0)
    ec = lax.broadcasted_iota(jnp.int32, (NUM_EXPERTS, NUM_EXPERTS), 1)
    excl = (er < ec).astype(jnp.float32)
    poff = lax.dot_general(
        pe, excl, (((1,), (0,)), ((), ())),
        preferred_element_type=jnp.float32,
    )

    slot = poff + csum
    pos0 = jnp.sum(one1 * slot, axis=1, keepdims=True)
    pos1 = jnp.sum(one2 * slot, axis=1, keepdims=True)

    lane = lax.broadcasted_iota(jnp.int32, (TOKENS, 128), 1)
    posm = jnp.where(lane == 0, pos0, jnp.where(lane == 1, pos1, 0.0))
    pos_ref[...] = posm.astype(jnp.int32)
    w_ref[...] = jnp.where(lane < LANES, w1, jnp.where(lane < 2 * LANES, w2, 0.0))

    jrow = lax.broadcasted_iota(jnp.int32, (128, NUM_EXPERTS), 0)
    started = (jnp.broadcast_to(poff, (128, NUM_EXPERTS))
               <= (jrow * TILE).astype(jnp.float32)).astype(jnp.float32)
    ones = jnp.ones((NUM_EXPERTS, 128), jnp.float32)
    te = lax.dot_general(
        started, ones, (((1,), (0,)), ((), ())),
        preferred_element_type=jnp.float32,
    ) - 1.0
    te_ref[...] = te.astype(jnp.int32)


def _mlp_body(te_ref, pos_ref, h_ref, gw_ref, dw_ref, o_ref):
    i = pl.program_id(0)
    pos0 = pos_ref[:, :1]
    pos1 = pos_ref[:, 1:2]
    slot = lax.broadcasted_iota(jnp.int32, (TOKENS, TILE), 1) + i * TILE
    sel = ((slot == pos0) | (slot == pos1)).astype(jnp.bfloat16)
    x = lax.dot_general(
        sel, h_ref[...], (((0,), (0,)), ((), ())),
        preferred_element_type=jnp.float32,
    ).astype(jnp.bfloat16)
    gu = lax.dot_general(
        x, gw_ref[0], (((1,), (1,)), ((), ())),
        preferred_element_type=jnp.float32,
    )
    gate = gu[:, :INTER]
    up = gu[:, INTER:]
    hmid = (gate * lax.logistic(gate) * up).astype(jnp.bfloat16)
    o_ref[...] = lax.dot_general(
        hmid, dw_ref[0], (((1,), (1,)), ((), ())),
        preferred_element_type=jnp.float32,
    )


TPW = TOKENS // NW
CHT = TPW // 2


@functools.partial(
    pl.kernel,
    out_type=jax.ShapeDtypeStruct((TOKENS, HIDDEN), jnp.float32),
    mesh=_mesh,
    scratch_types=[
        pltpu.VMEM((2, 2 * CHT), jnp.int32),
        pltpu.VMEM((TPW, 128), jnp.float32),
        pltpu.VMEM((2 * CHT, HIDDEN), jnp.float32),
        pltpu.VMEM((CHT, HIDDEN), jnp.float32),
        pltpu.SemaphoreType.DMA,
    ],
)
def _combine(pos_hbm, w_hbm, y_hbm, out_hbm, idx_v, w_v, buf_v, o_v, sem):
    base = _wid() * TPW
    pltpu.sync_copy(pos_hbm.at[pl.ds(2 * base, 2 * CHT)], idx_v.at[0])
    pltpu.sync_copy(pos_hbm.at[pl.ds(2 * base + 2 * CHT, 2 * CHT)],
                    idx_v.at[1])
    pltpu.sync_copy(w_hbm.at[pl.ds(base, TPW)], w_v)
    for ch in range(2):
        for c in range((2 * CHT) // LANES):
            v = idx_v[ch, pl.ds(c * LANES, LANES)]
            idx_v[ch, pl.ds(c * LANES, LANES)] = jnp.minimum(
                jnp.maximum(v, 0), PAD - 1)
        pltpu.async_copy(y_hbm.at[idx_v.at[ch]], buf_v, sem).wait()
        for j in range(CHT):
            p = ch * CHT + j
            w0 = w_v[p, pl.ds(0, LANES)]
            w1 = w_v[p, pl.ds(LANES, LANES)]

            def _col(cb, carry, j=j, w0=w0, w1=w1):
                s = cb * 64
                for u in range(4):
                    sl = pl.ds(s + u * LANES, LANES)
                    o_v[j, sl] = buf_v[2 * j, sl] * w0 + buf_v[2 * j + 1, sl] * w1
                return carry

            lax.fori_loop(0, HIDDEN // 64, _col, 0)
        pltpu.sync_copy(o_v, out_hbm.at[pl.ds(base + ch * CHT, CHT)])


@jax.jit
def kernel(hidden_states, router, gate_up_proj, down_proj):
    pos_pad, w_pad, te_pad = pl.pallas_call(
        _route_sort_body,
        out_shape=[
            jax.ShapeDtypeStruct((TOKENS, 128), jnp.int32),
            jax.ShapeDtypeStruct((TOKENS, 128), jnp.float32),
            jax.ShapeDtypeStruct((128, 128), jnp.int32),
        ],
    )(hidden_states, router)

    pos_flat = pos_pad[:, :TOP_K].reshape(NPAIR)
    te_arr = te_pad[:NT, 0]

    hb = hidden_states.astype(jnp.bfloat16)
    gw = gate_up_proj.astype(jnp.bfloat16)
    dw = down_proj.astype(jnp.bfloat16)
    y_sorted = pl.pallas_call(
        _mlp_body,
        grid_spec=pltpu.PrefetchScalarGridSpec(
            num_scalar_prefetch=1,
            grid=(NT,),
            in_specs=[
                pl.BlockSpec((TOKENS, 128), lambda i, te: (0, 0)),
                pl.BlockSpec((TOKENS, HIDDEN), lambda i, te: (0, 0)),
                pl.BlockSpec((1, 2 * INTER, HIDDEN),
                             lambda i, te: (te[i], 0, 0)),
                pl.BlockSpec((1, HIDDEN, INTER),
                             lambda i, te: (te[i], 0, 0)),
            ],
            out_specs=pl.BlockSpec((TILE, HIDDEN), lambda i, te: (i, 0)),
        ),
        out_shape=jax.ShapeDtypeStruct((PAD, HIDDEN), jnp.float32),
    )(te_arr, pos_pad, hb, gw, dw)

    return _combine(pos_flat, w_pad, y_sorted)


# speedup vs baseline: 1.3302x; 1.3302x over previous
import functools

import jax
import jax.numpy as jnp
from jax import lax
from jax.experimental import pallas as pl
from jax.experimental.pallas import tpu as pltpu
from jax.experimental.pallas import tpu_sc as plsc

NUM_EXPERTS = 8
TOP_K = 2
HIDDEN = 1024
INTER = 2048
TOKENS = 2048

TILE = 256
PAD = 6144
NT = PAD // TILE
NPAIR = TOKENS * TOP_K

NC = 2
NS = 16
NW = NC * NS
LANES = 16

_mesh = plsc.VectorSubcoreMesh(core_axis_name="c", subcore_axis_name="s")


def _wid():
    return lax.axis_index("s") * NC + lax.axis_index("c")


def _route_sort_body(h_ref, r_ref, pos_ref, w_ref, te_ref):
    h = h_ref[...]
    r = r_ref[...]
    logits = lax.dot_general(
        h, r, (((1,), (1,)), ((), ())),
        preferred_element_type=jnp.float32,
    )
    iota8 = lax.broadcasted_iota(jnp.int32, logits.shape, 1)
    m1 = jnp.max(logits, axis=1, keepdims=True)
    i1 = jnp.min(jnp.where(logits == m1, iota8, NUM_EXPERTS), axis=1,
                 keepdims=True)
    masked = jnp.where(iota8 == i1, -jnp.inf, logits)
    m2 = jnp.max(masked, axis=1, keepdims=True)
    i2 = jnp.min(jnp.where(masked == m2, iota8, NUM_EXPERTS), axis=1,
                 keepdims=True)
    w1 = 1.0 / (1.0 + jnp.exp(m2 - m1))
    w2 = 1.0 - w1

    one1 = (iota8 == i1).astype(jnp.float32)
    one2 = (iota8 == i2).astype(jnp.float32)
    occ = one1 + one2

    rr = lax.broadcasted_iota(jnp.int32, (TOKENS, TOKENS), 0)
    cc = lax.broadcasted_iota(jnp.int32, (TOKENS, TOKENS), 1)
    ltri = (cc < rr).astype(jnp.float32)
    csum = lax.dot_general(
        ltri, occ, (((1,), (0,)), ((), ())),
        preferred_element_type=jnp.float32,
    )

    n_tot = jnp.sum(occ, axis=0, keepdims=True)
    pe = jnp.floor((n_tot + (TILE - 1)) * (1.0 / TILE)) * TILE
    er = lax.broadcasted_iota(jnp.int32, (NUM_EXPERTS, NUM_EXPERTS), 0)
    ec = lax.broadcasted_iota(jnp.int32, (NUM_EXPERTS, NUM_EXPERTS), 1)
    excl = (er < ec).astype(jnp.float32)
    poff = lax.dot_general(
        pe, excl, (((1,), (0,)), ((), ())),
        preferred_element_type=jnp.float32,
    )

    slot = poff + csum
    pos0 = jnp.sum(one1 * slot, axis=1, keepdims=True)
    pos1 = jnp.sum(one2 * slot, axis=1, keepdims=True)

    lane = lax.broadcasted_iota(jnp.int32, (TOKENS, 128), 1)
    posm = jnp.where(lane == 0, pos0, jnp.where(lane == 1, pos1, 0.0))
    pos_ref[...] = posm.astype(jnp.int32)
    w_ref[...] = jnp.where(lane < LANES, w1, jnp.where(lane < 2 * LANES, w2, 0.0))

    jrow = lax.broadcasted_iota(jnp.int32, (128, NUM_EXPERTS), 0)
    started = (jnp.broadcast_to(poff, (128, NUM_EXPERTS))
               <= (jrow * TILE).astype(jnp.float32)).astype(jnp.float32)
    ones = jnp.ones((NUM_EXPERTS, 128), jnp.float32)
    te = lax.dot_general(
        started, ones, (((1,), (0,)), ((), ())),
        preferred_element_type=jnp.float32,
    ) - 1.0
    te_ref[...] = te.astype(jnp.int32)


def _mlp_body(te_ref, pos_ref, h_ref, gw_ref, dw_ref, o_ref):
    i = pl.program_id(0)
    pos0 = pos_ref[:, :1]
    pos1 = pos_ref[:, 1:2]
    slot = lax.broadcasted_iota(jnp.int32, (TOKENS, TILE), 1) + i * TILE
    sel = ((slot == pos0) | (slot == pos1)).astype(jnp.bfloat16)
    x = lax.dot_general(
        sel, h_ref[...], (((0,), (0,)), ((), ())),
        preferred_element_type=jnp.float32,
    ).astype(jnp.bfloat16)
    gu = lax.dot_general(
        x, gw_ref[0], (((1,), (1,)), ((), ())),
        preferred_element_type=jnp.float32,
    )
    gate = gu[:, :INTER]
    up = gu[:, INTER:]
    hmid = (gate * lax.logistic(gate) * up).astype(jnp.bfloat16)
    o_ref[...] = lax.dot_general(
        hmid, dw_ref[0], (((1,), (1,)), ((), ())),
        preferred_element_type=jnp.float32,
    )


TPW = TOKENS // NW
NCH = 4
CHT = TPW // NCH


@functools.partial(
    pl.kernel,
    out_type=jax.ShapeDtypeStruct((TOKENS, HIDDEN), jnp.float32),
    mesh=_mesh,
    scratch_types=[
        pltpu.VMEM((NCH, 2 * CHT), jnp.int32),
        pltpu.VMEM((TPW, 128), jnp.float32),
        pltpu.VMEM((2 * CHT, HIDDEN), jnp.float32),
        pltpu.VMEM((2 * CHT, HIDDEN), jnp.float32),
        pltpu.VMEM((CHT, HIDDEN), jnp.float32),
        pltpu.SemaphoreType.DMA,
        pltpu.SemaphoreType.DMA,
    ],
)
def _combine(pos_hbm, w_hbm, y_hbm, out_hbm, idx_v, w_v, buf0_v, buf1_v, o_v,
             sem0, sem1):
    base = _wid() * TPW
    for ch in range(NCH):
        pltpu.sync_copy(pos_hbm.at[pl.ds(2 * base + ch * 2 * CHT, 2 * CHT)],
                        idx_v.at[ch])
    pltpu.sync_copy(w_hbm.at[pl.ds(base, TPW)], w_v)
    for ch in range(NCH):
        for c in range((2 * CHT) // LANES):
            v = idx_v[ch, pl.ds(c * LANES, LANES)]
            idx_v[ch, pl.ds(c * LANES, LANES)] = jnp.minimum(
                jnp.maximum(v, 0), PAD - 1)
    bufs = (buf0_v, buf1_v)
    sems = (sem0, sem1)
    cps = [pltpu.async_copy(y_hbm.at[idx_v.at[ch]], bufs[ch], sems[ch])
           for ch in range(2)]
    for ch in range(NCH):
        cps[ch].wait()
        buf_v = bufs[ch % 2]
        for j in range(CHT):
            p = ch * CHT + j
            w0 = w_v[p, pl.ds(0, LANES)]
            w1 = w_v[p, pl.ds(LANES, LANES)]

            def _col(cb, carry, j=j, w0=w0, w1=w1, buf_v=buf_v):
                s = cb * 64
                for u in range(4):
                    sl = pl.ds(s + u * LANES, LANES)
                    o_v[j, sl] = buf_v[2 * j, sl] * w0 + buf_v[2 * j + 1, sl] * w1
                return carry

            lax.fori_loop(0, HIDDEN // 64, _col, 0)
        pltpu.sync_copy(o_v, out_hbm.at[pl.ds(base + ch * CHT, CHT)])
        if ch + 2 < NCH:
            cps.append(pltpu.async_copy(
                y_hbm.at[idx_v.at[ch + 2]], bufs[ch % 2], sems[ch % 2]))


@jax.jit
def kernel(hidden_states, router, gate_up_proj, down_proj):
    pos_pad, w_pad, te_pad = pl.pallas_call(
        _route_sort_body,
        out_shape=[
            jax.ShapeDtypeStruct((TOKENS, 128), jnp.int32),
            jax.ShapeDtypeStruct((TOKENS, 128), jnp.float32),
            jax.ShapeDtypeStruct((128, 128), jnp.int32),
        ],
    )(hidden_states, router)

    pos_flat = pos_pad[:, :TOP_K].reshape(NPAIR)
    te_arr = te_pad[:NT, 0]

    hb = hidden_states.astype(jnp.bfloat16)
    gw = gate_up_proj.astype(jnp.bfloat16)
    dw = down_proj.astype(jnp.bfloat16)
    y_sorted = pl.pallas_call(
        _mlp_body,
        grid_spec=pltpu.PrefetchScalarGridSpec(
            num_scalar_prefetch=1,
            grid=(NT,),
            in_specs=[
                pl.BlockSpec((TOKENS, 128), lambda i, te: (0, 0)),
                pl.BlockSpec((TOKENS, HIDDEN), lambda i, te: (0, 0)),
                pl.BlockSpec((1, 2 * INTER, HIDDEN),
                             lambda i, te: (te[i], 0, 0)),
                pl.BlockSpec((1, HIDDEN, INTER),
                             lambda i, te: (te[i], 0, 0)),
            ],
            out_specs=pl.BlockSpec((TILE, HIDDEN), lambda i, te: (i, 0)),
        ),
        out_shape=jax.ShapeDtypeStruct((PAD, HIDDEN), jnp.float32),
    )(te_arr, pos_pad, hb, gw, dw)

    return _combine(pos_flat, w_pad, y_sorted)


# speedup vs baseline: 1.4450x; 1.0863x over previous
import functools

import jax
import jax.numpy as jnp
from jax import lax
from jax.experimental import pallas as pl
from jax.experimental.pallas import tpu as pltpu
from jax.experimental.pallas import tpu_sc as plsc

NUM_EXPERTS = 8
TOP_K = 2
HIDDEN = 1024
INTER = 2048
TOKENS = 2048

TILE = 256
PAD = 6144
NT = PAD // TILE
NPAIR = TOKENS * TOP_K

NC = 2
NS = 16
NW = NC * NS
LANES = 16

_mesh = plsc.VectorSubcoreMesh(core_axis_name="c", subcore_axis_name="s")


def _wid():
    return lax.axis_index("s") * NC + lax.axis_index("c")


def _route_sort_body(h_ref, r_ref, pos_ref, w_ref, te_ref):
    h = h_ref[...]
    r = r_ref[...]
    logits = lax.dot_general(
        h, r, (((1,), (1,)), ((), ())),
        preferred_element_type=jnp.float32,
    )
    iota8 = lax.broadcasted_iota(jnp.int32, logits.shape, 1)
    m1 = jnp.max(logits, axis=1, keepdims=True)
    i1 = jnp.min(jnp.where(logits == m1, iota8, NUM_EXPERTS), axis=1,
                 keepdims=True)
    masked = jnp.where(iota8 == i1, -jnp.inf, logits)
    m2 = jnp.max(masked, axis=1, keepdims=True)
    i2 = jnp.min(jnp.where(masked == m2, iota8, NUM_EXPERTS), axis=1,
                 keepdims=True)
    w1 = 1.0 / (1.0 + jnp.exp(m2 - m1))
    w2 = 1.0 - w1

    one1 = (iota8 == i1).astype(jnp.float32)
    one2 = (iota8 == i2).astype(jnp.float32)
    occ = one1 + one2

    rr = lax.broadcasted_iota(jnp.int32, (TOKENS, TOKENS), 0)
    cc = lax.broadcasted_iota(jnp.int32, (TOKENS, TOKENS), 1)
    ltri = (cc < rr).astype(jnp.float32)
    csum = lax.dot_general(
        ltri, occ, (((1,), (0,)), ((), ())),
        preferred_element_type=jnp.float32,
    )

    n_tot = jnp.sum(occ, axis=0, keepdims=True)
    pe = jnp.floor((n_tot + (TILE - 1)) * (1.0 / TILE)) * TILE
    er = lax.broadcasted_iota(jnp.int32, (NUM_EXPERTS, NUM_EXPERTS), 0)
    ec = lax.broadcasted_iota(jnp.int32, (NUM_EXPERTS, NUM_EXPERTS), 1)
    excl = (er < ec).astype(jnp.float32)
    poff = lax.dot_general(
        pe, excl, (((1,), (0,)), ((), ())),
        preferred_element_type=jnp.float32,
    )

    slot = poff + csum
    pos0 = jnp.sum(one1 * slot, axis=1, keepdims=True)
    pos1 = jnp.sum(one2 * slot, axis=1, keepdims=True)

    lane = lax.broadcasted_iota(jnp.int32, (TOKENS, 128), 1)
    posm = jnp.where(lane == 0, pos0, jnp.where(lane == 1, pos1, 0.0))
    pos_ref[...] = posm.astype(jnp.int32)
    w_ref[...] = jnp.where(lane < LANES, w1, jnp.where(lane < 2 * LANES, w2, 0.0))

    jrow = lax.broadcasted_iota(jnp.int32, (128, NUM_EXPERTS), 0)
    started = (jnp.broadcast_to(poff, (128, NUM_EXPERTS))
               <= (jrow * TILE).astype(jnp.float32)).astype(jnp.float32)
    ones = jnp.ones((NUM_EXPERTS, 128), jnp.float32)
    te = lax.dot_general(
        started, ones, (((1,), (0,)), ((), ())),
        preferred_element_type=jnp.float32,
    ) - 1.0
    te_ref[...] = te.astype(jnp.int32)


def _mlp_body(te_ref, pos_ref, h_ref, gw_ref, dw_ref, o_ref, dwb_ref):
    i = pl.program_id(0)
    changed = (i == 0) | (te_ref[i] != te_ref[jnp.maximum(i - 1, 0)])

    @pl.when(changed)
    def _cast_down():
        dwb_ref[...] = dw_ref[0].astype(jnp.bfloat16)

    pos0 = pos_ref[:, :1]
    pos1 = pos_ref[:, 1:2]
    slot = lax.broadcasted_iota(jnp.int32, (TOKENS, TILE), 1) + i * TILE
    sel = ((slot == pos0) | (slot == pos1)).astype(jnp.bfloat16)
    x = lax.dot_general(
        sel, h_ref[...], (((0,), (0,)), ((), ())),
        preferred_element_type=jnp.float32,
    ).astype(jnp.bfloat16)
    gu = lax.dot_general(
        x, gw_ref[0], (((1,), (1,)), ((), ())),
        preferred_element_type=jnp.float32,
    )
    gate = gu[:, :INTER]
    up = gu[:, INTER:]
    hmid = (gate * lax.logistic(gate) * up).astype(jnp.bfloat16)
    o_ref[...] = lax.dot_general(
        hmid, dwb_ref[...], (((1,), (1,)), ((), ())),
        preferred_element_type=jnp.float32,
    )


TPW = TOKENS // NW
NCH = 4
CHT = TPW // NCH


@functools.partial(
    pl.kernel,
    out_type=jax.ShapeDtypeStruct((TOKENS, HIDDEN), jnp.float32),
    mesh=_mesh,
    scratch_types=[
        pltpu.VMEM((NCH, 2 * CHT), jnp.int32),
        pltpu.VMEM((TPW, 128), jnp.float32),
        pltpu.VMEM((2 * CHT, HIDDEN), jnp.float32),
        pltpu.VMEM((2 * CHT, HIDDEN), jnp.float32),
        pltpu.VMEM((CHT, HIDDEN), jnp.float32),
        pltpu.SemaphoreType.DMA,
        pltpu.SemaphoreType.DMA,
    ],
)
def _combine(pos_hbm, w_hbm, y_hbm, out_hbm, idx_v, w_v, buf0_v, buf1_v, o_v,
             sem0, sem1):
    base = _wid() * TPW
    for ch in range(NCH):
        pltpu.sync_copy(pos_hbm.at[pl.ds(2 * base + ch * 2 * CHT, 2 * CHT)],
                        idx_v.at[ch])
    pltpu.sync_copy(w_hbm.at[pl.ds(base, TPW)], w_v)
    for ch in range(NCH):
        for c in range((2 * CHT) // LANES):
            v = idx_v[ch, pl.ds(c * LANES, LANES)]
            idx_v[ch, pl.ds(c * LANES, LANES)] = jnp.minimum(
                jnp.maximum(v, 0), PAD - 1)
    bufs = (buf0_v, buf1_v)
    sems = (sem0, sem1)
    cps = [pltpu.async_copy(y_hbm.at[idx_v.at[ch]], bufs[ch], sems[ch])
           for ch in range(2)]
    for ch in range(NCH):
        cps[ch].wait()
        buf_v = bufs[ch % 2]
        for j in range(CHT):
            p = ch * CHT + j
            w0 = w_v[p, pl.ds(0, LANES)]
            w1 = w_v[p, pl.ds(LANES, LANES)]

            def _col(cb, carry, j=j, w0=w0, w1=w1, buf_v=buf_v):
                s = cb * 64
                for u in range(4):
                    sl = pl.ds(s + u * LANES, LANES)
                    o_v[j, sl] = buf_v[2 * j, sl] * w0 + buf_v[2 * j + 1, sl] * w1
                return carry

            lax.fori_loop(0, HIDDEN // 64, _col, 0)
        pltpu.sync_copy(o_v, out_hbm.at[pl.ds(base + ch * CHT, CHT)])
        if ch + 2 < NCH:
            cps.append(pltpu.async_copy(
                y_hbm.at[idx_v.at[ch + 2]], bufs[ch % 2], sems[ch % 2]))


@jax.jit
def kernel(hidden_states, router, gate_up_proj, down_proj):
    pos_pad, w_pad, te_pad = pl.pallas_call(
        _route_sort_body,
        out_shape=[
            jax.ShapeDtypeStruct((TOKENS, 128), jnp.int32),
            jax.ShapeDtypeStruct((TOKENS, 128), jnp.float32),
            jax.ShapeDtypeStruct((128, 128), jnp.int32),
        ],
    )(hidden_states, router)

    pos_flat = pos_pad[:, :TOP_K].reshape(NPAIR)
    te_arr = te_pad[:NT, 0]

    hb = hidden_states.astype(jnp.bfloat16)
    gw = gate_up_proj.astype(jnp.bfloat16)
    y_sorted = pl.pallas_call(
        _mlp_body,
        grid_spec=pltpu.PrefetchScalarGridSpec(
            num_scalar_prefetch=1,
            grid=(NT,),
            in_specs=[
                pl.BlockSpec((TOKENS, 128), lambda i, te: (0, 0)),
                pl.BlockSpec((TOKENS, HIDDEN), lambda i, te: (0, 0)),
                pl.BlockSpec((1, 2 * INTER, HIDDEN),
                             lambda i, te: (te[i], 0, 0)),
                pl.BlockSpec((1, HIDDEN, INTER),
                             lambda i, te: (te[i], 0, 0)),
            ],
            out_specs=pl.BlockSpec((TILE, HIDDEN), lambda i, te: (i, 0)),
            scratch_shapes=[pltpu.VMEM((HIDDEN, INTER), jnp.bfloat16)],
        ),
        out_shape=jax.ShapeDtypeStruct((PAD, HIDDEN), jnp.float32),
    )(te_arr, pos_pad, hb, gw, down_proj)

    return _combine(pos_flat, w_pad, y_sorted)


# speedup vs baseline: 1.4899x; 1.0311x over previous
import functools

import jax
import jax.numpy as jnp
from jax import lax
from jax.experimental import pallas as pl
from jax.experimental.pallas import tpu as pltpu
from jax.experimental.pallas import tpu_sc as plsc

NUM_EXPERTS = 8
TOP_K = 2
HIDDEN = 1024
INTER = 2048
TOKENS = 2048

TILE = 256
PAD = 6144
NT = PAD // TILE
NPAIR = TOKENS * TOP_K

NC = 2
NS = 16
NW = NC * NS
LANES = 16

_mesh = plsc.VectorSubcoreMesh(core_axis_name="c", subcore_axis_name="s")


def _wid():
    return lax.axis_index("s") * NC + lax.axis_index("c")


def _route_sort_body(h_ref, r_ref, pos_ref, w_ref, te_ref):
    h = h_ref[...]
    r = r_ref[...]
    logits = lax.dot_general(
        h, r, (((1,), (1,)), ((), ())),
        preferred_element_type=jnp.float32,
    )
    iota8 = lax.broadcasted_iota(jnp.int32, logits.shape, 1)
    m1 = jnp.max(logits, axis=1, keepdims=True)
    i1 = jnp.min(jnp.where(logits == m1, iota8, NUM_EXPERTS), axis=1,
                 keepdims=True)
    masked = jnp.where(iota8 == i1, -jnp.inf, logits)
    m2 = jnp.max(masked, axis=1, keepdims=True)
    i2 = jnp.min(jnp.where(masked == m2, iota8, NUM_EXPERTS), axis=1,
                 keepdims=True)
    w1 = 1.0 / (1.0 + jnp.exp(m2 - m1))
    w2 = 1.0 - w1

    one1 = (iota8 == i1).astype(jnp.float32)
    one2 = (iota8 == i2).astype(jnp.float32)
    occ = one1 + one2

    rr = lax.broadcasted_iota(jnp.int32, (TOKENS, TOKENS), 0)
    cc = lax.broadcasted_iota(jnp.int32, (TOKENS, TOKENS), 1)
    ltri = (cc < rr).astype(jnp.float32)
    csum = lax.dot_general(
        ltri, occ, (((1,), (0,)), ((), ())),
        preferred_element_type=jnp.float32,
    )

    n_tot = jnp.sum(occ, axis=0, keepdims=True)
    pe = jnp.floor((n_tot + (TILE - 1)) * (1.0 / TILE)) * TILE
    er = lax.broadcasted_iota(jnp.int32, (NUM_EXPERTS, NUM_EXPERTS), 0)
    ec = lax.broadcasted_iota(jnp.int32, (NUM_EXPERTS, NUM_EXPERTS), 1)
    excl = (er < ec).astype(jnp.float32)
    poff = lax.dot_general(
        pe, excl, (((1,), (0,)), ((), ())),
        preferred_element_type=jnp.float32,
    )

    slot = poff + csum
    pos0 = jnp.sum(one1 * slot, axis=1, keepdims=True)
    pos1 = jnp.sum(one2 * slot, axis=1, keepdims=True)

    lane = lax.broadcasted_iota(jnp.int32, (TOKENS, 128), 1)
    posm = jnp.where(lane == 0, pos0, jnp.where(lane == 1, pos1, 0.0))
    pos_ref[...] = posm.astype(jnp.int32)
    w_ref[...] = jnp.where(lane < LANES, w1, jnp.where(lane < 2 * LANES, w2, 0.0))

    jrow = lax.broadcasted_iota(jnp.int32, (128, NUM_EXPERTS), 0)
    started = (jnp.broadcast_to(poff, (128, NUM_EXPERTS))
               <= (jrow * TILE).astype(jnp.float32)).astype(jnp.float32)
    ones = jnp.ones((NUM_EXPERTS, 128), jnp.float32)
    te = lax.dot_general(
        started, ones, (((1,), (0,)), ((), ())),
        preferred_element_type=jnp.float32,
    ) - 1.0
    te_ref[...] = te.astype(jnp.int32)


def _mlp_up_body(te_ref, pos_ref, h_ref, gw_ref, hm_ref, gwb_ref):
    i = pl.program_id(0)
    changed = (i == 0) | (te_ref[i] != te_ref[jnp.maximum(i - 1, 0)])

    @pl.when(changed)
    def _cast_gw():
        gwb_ref[...] = gw_ref[0].astype(jnp.bfloat16)

    pos0 = pos_ref[:, :1]
    pos1 = pos_ref[:, 1:2]
    slot = lax.broadcasted_iota(jnp.int32, (TOKENS, TILE), 1) + i * TILE
    sel = ((slot == pos0) | (slot == pos1)).astype(jnp.bfloat16)
    x = lax.dot_general(
        sel, h_ref[...], (((0,), (0,)), ((), ())),
        preferred_element_type=jnp.float32,
    ).astype(jnp.bfloat16)
    gu = lax.dot_general(
        x, gwb_ref[...], (((1,), (1,)), ((), ())),
        preferred_element_type=jnp.float32,
    )
    gate = gu[:, :INTER]
    up = gu[:, INTER:]
    hm_ref[...] = (gate * lax.logistic(gate) * up).astype(jnp.bfloat16)


def _mlp_down_body(te_ref, hm_ref, dw_ref, o_ref, dwb_ref):
    i = pl.program_id(0)
    changed = (i == 0) | (te_ref[i] != te_ref[jnp.maximum(i - 1, 0)])

    @pl.when(changed)
    def _cast_down():
        dwb_ref[...] = dw_ref[0].astype(jnp.bfloat16)

    o_ref[...] = lax.dot_general(
        hm_ref[...], dwb_ref[...], (((1,), (1,)), ((), ())),
        preferred_element_type=jnp.float32,
    )


TPW = TOKENS // NW
NCH = 4
CHT = TPW // NCH


@functools.partial(
    pl.kernel,
    out_type=jax.ShapeDtypeStruct((TOKENS, HIDDEN), jnp.float32),
    mesh=_mesh,
    scratch_types=[
        pltpu.VMEM((NCH, 2 * CHT), jnp.int32),
        pltpu.VMEM((TPW, 128), jnp.float32),
        pltpu.VMEM((2 * CHT, HIDDEN), jnp.float32),
        pltpu.VMEM((2 * CHT, HIDDEN), jnp.float32),
        pltpu.VMEM((CHT, HIDDEN), jnp.float32),
        pltpu.SemaphoreType.DMA,
        pltpu.SemaphoreType.DMA,
    ],
)
def _combine(pos_hbm, w_hbm, y_hbm, out_hbm, idx_v, w_v, buf0_v, buf1_v, o_v,
             sem0, sem1):
    base = _wid() * TPW
    for ch in range(NCH):
        pltpu.sync_copy(pos_hbm.at[pl.ds(2 * base + ch * 2 * CHT, 2 * CHT)],
                        idx_v.at[ch])
    pltpu.sync_copy(w_hbm.at[pl.ds(base, TPW)], w_v)
    for ch in range(NCH):
        for c in range((2 * CHT) // LANES):
            v = idx_v[ch, pl.ds(c * LANES, LANES)]
            idx_v[ch, pl.ds(c * LANES, LANES)] = jnp.minimum(
                jnp.maximum(v, 0), PAD - 1)
    bufs = (buf0_v, buf1_v)
    sems = (sem0, sem1)
    cps = [pltpu.async_copy(y_hbm.at[idx_v.at[ch]], bufs[ch], sems[ch])
           for ch in range(2)]
    for ch in range(NCH):
        cps[ch].wait()
        buf_v = bufs[ch % 2]
        for j in range(CHT):
            p = ch * CHT + j
            w0 = w_v[p, pl.ds(0, LANES)]
            w1 = w_v[p, pl.ds(LANES, LANES)]

            def _col(cb, carry, j=j, w0=w0, w1=w1, buf_v=buf_v):
                s = cb * 64
                for u in range(4):
                    sl = pl.ds(s + u * LANES, LANES)
                    o_v[j, sl] = buf_v[2 * j, sl] * w0 + buf_v[2 * j + 1, sl] * w1
                return carry

            lax.fori_loop(0, HIDDEN // 64, _col, 0)
        pltpu.sync_copy(o_v, out_hbm.at[pl.ds(base + ch * CHT, CHT)])
        if ch + 2 < NCH:
            cps.append(pltpu.async_copy(
                y_hbm.at[idx_v.at[ch + 2]], bufs[ch % 2], sems[ch % 2]))


@jax.jit
def kernel(hidden_states, router, gate_up_proj, down_proj):
    pos_pad, w_pad, te_pad = pl.pallas_call(
        _route_sort_body,
        out_shape=[
            jax.ShapeDtypeStruct((TOKENS, 128), jnp.int32),
            jax.ShapeDtypeStruct((TOKENS, 128), jnp.float32),
            jax.ShapeDtypeStruct((128, 128), jnp.int32),
        ],
    )(hidden_states, router)

    pos_flat = pos_pad[:, :TOP_K].reshape(NPAIR)
    te_arr = te_pad[:NT, 0]

    hb = hidden_states.astype(jnp.bfloat16)
    hmid_sorted = pl.pallas_call(
        _mlp_up_body,
        grid_spec=pltpu.PrefetchScalarGridSpec(
            num_scalar_prefetch=1,
            grid=(NT,),
            in_specs=[
                pl.BlockSpec((TOKENS, 128), lambda i, te: (0, 0)),
                pl.BlockSpec((TOKENS, HIDDEN), lambda i, te: (0, 0)),
                pl.BlockSpec((1, 2 * INTER, HIDDEN),
                             lambda i, te: (te[i], 0, 0)),
            ],
            out_specs=pl.BlockSpec((TILE, INTER), lambda i, te: (i, 0)),
            scratch_shapes=[pltpu.VMEM((2 * INTER, HIDDEN), jnp.bfloat16)],
        ),
        out_shape=jax.ShapeDtypeStruct((PAD, INTER), jnp.bfloat16),
    )(te_arr, pos_pad, hb, gate_up_proj)

    y_sorted = pl.pallas_call(
        _mlp_down_body,
        grid_spec=pltpu.PrefetchScalarGridSpec(
            num_scalar_prefetch=1,
            grid=(NT,),
            in_specs=[
                pl.BlockSpec((TILE, INTER), lambda i, te: (i, 0)),
                pl.BlockSpec((1, HIDDEN, INTER),
                             lambda i, te: (te[i], 0, 0)),
            ],
            out_specs=pl.BlockSpec((TILE, HIDDEN), lambda i, te: (i, 0)),
            scratch_shapes=[pltpu.VMEM((HIDDEN, INTER), jnp.bfloat16)],
        ),
        out_shape=jax.ShapeDtypeStruct((PAD, HIDDEN), jnp.float32),
    )(te_arr, hmid_sorted, down_proj)

    return _combine(pos_flat, w_pad, y_sorted)


# speedup vs baseline: 1.6218x; 1.0885x over previous
import functools

import jax
import jax.numpy as jnp
from jax import lax
from jax.experimental import pallas as pl
from jax.experimental.pallas import tpu as pltpu
from jax.experimental.pallas import tpu_sc as plsc

NUM_EXPERTS = 8
TOP_K = 2
HIDDEN = 1024
INTER = 2048
TOKENS = 2048

TILE = 256
PAD = 6144
NT = PAD // TILE
NPAIR = TOKENS * TOP_K

NC = 2
NS = 16
NW = NC * NS
LANES = 16

_mesh = plsc.VectorSubcoreMesh(core_axis_name="c", subcore_axis_name="s")


def _wid():
    return lax.axis_index("s") * NC + lax.axis_index("c")


def _route_sort_body(h_ref, r_ref, pos_ref, w_ref, te_ref, lv_ref):
    h = h_ref[...]
    r = r_ref[...]
    logits = lax.dot_general(
        h, r, (((1,), (1,)), ((), ())),
        preferred_element_type=jnp.float32,
    )
    iota8 = lax.broadcasted_iota(jnp.int32, logits.shape, 1)
    m1 = jnp.max(logits, axis=1, keepdims=True)
    i1 = jnp.min(jnp.where(logits == m1, iota8, NUM_EXPERTS), axis=1,
                 keepdims=True)
    masked = jnp.where(iota8 == i1, -jnp.inf, logits)
    m2 = jnp.max(masked, axis=1, keepdims=True)
    i2 = jnp.min(jnp.where(masked == m2, iota8, NUM_EXPERTS), axis=1,
                 keepdims=True)
    w1 = 1.0 / (1.0 + jnp.exp(m2 - m1))
    w2 = 1.0 - w1

    one1 = (iota8 == i1).astype(jnp.float32)
    one2 = (iota8 == i2).astype(jnp.float32)
    occ = one1 + one2

    rr = lax.broadcasted_iota(jnp.int32, (TOKENS, TOKENS), 0)
    cc = lax.broadcasted_iota(jnp.int32, (TOKENS, TOKENS), 1)
    ltri = (cc < rr).astype(jnp.float32)
    csum = lax.dot_general(
        ltri, occ, (((1,), (0,)), ((), ())),
        preferred_element_type=jnp.float32,
    )

    n_tot = jnp.sum(occ, axis=0, keepdims=True)
    pe = jnp.floor((n_tot + (TILE - 1)) * (1.0 / TILE)) * TILE
    er = lax.broadcasted_iota(jnp.int32, (NUM_EXPERTS, NUM_EXPERTS), 0)
    ec = lax.broadcasted_iota(jnp.int32, (NUM_EXPERTS, NUM_EXPERTS), 1)
    excl = (er < ec).astype(jnp.float32)
    poff = lax.dot_general(
        pe, excl, (((1,), (0,)), ((), ())),
        preferred_element_type=jnp.float32,
    )

    slot = poff + csum
    pos0 = jnp.sum(one1 * slot, axis=1, keepdims=True)
    pos1 = jnp.sum(one2 * slot, axis=1, keepdims=True)

    lane = lax.broadcasted_iota(jnp.int32, (TOKENS, 128), 1)
    posm = jnp.where(lane == 0, pos0, jnp.where(lane == 1, pos1, 0.0))
    pos_ref[...] = posm.astype(jnp.int32)
    w_ref[...] = jnp.where(lane < LANES, w1, jnp.where(lane < 2 * LANES, w2, 0.0))

    jrow = lax.broadcasted_iota(jnp.int32, (128, NUM_EXPERTS), 0)
    started = (jnp.broadcast_to(poff, (128, NUM_EXPERTS))
               <= (jrow * TILE).astype(jnp.float32)).astype(jnp.float32)
    ones = jnp.ones((NUM_EXPERTS, 128), jnp.float32)
    te = lax.dot_general(
        started, ones, (((1,), (0,)), ((), ())),
        preferred_element_type=jnp.float32,
    ) - 1.0
    te_ref[...] = te.astype(jnp.int32)

    total = jnp.sum(pe, axis=1, keepdims=True)
    ts = lax.broadcasted_iota(jnp.int32, (128, 128), 0) * TILE
    lv_ref[...] = (ts.astype(jnp.float32)
                   < jnp.broadcast_to(total, (128, 128))).astype(jnp.int32)


def _mlp_up_body(te_ref, lv_ref, pos_ref, h_ref, gw_ref, hm_ref, gwb_ref):
    i = pl.program_id(0)
    changed = (i == 0) | (te_ref[i] != te_ref[jnp.maximum(i - 1, 0)])

    @pl.when(changed)
    def _cast_gw():
        gwb_ref[...] = gw_ref[0].astype(jnp.bfloat16)

    @pl.when(lv_ref[i] > 0)
    def _compute():
        pos0 = pos_ref[:, :1]
        pos1 = pos_ref[:, 1:2]
        slot = lax.broadcasted_iota(jnp.int32, (TOKENS, TILE), 1) + i * TILE
        sel = ((slot == pos0) | (slot == pos1)).astype(jnp.bfloat16)
        x = lax.dot_general(
            sel, h_ref[...], (((0,), (0,)), ((), ())),
            preferred_element_type=jnp.float32,
        ).astype(jnp.bfloat16)
        gu = lax.dot_general(
            x, gwb_ref[...], (((1,), (1,)), ((), ())),
            preferred_element_type=jnp.float32,
        )
        gate = gu[:, :INTER]
        up = gu[:, INTER:]
        hm_ref[...] = (gate * lax.logistic(gate) * up).astype(jnp.bfloat16)


def _mlp_down_body(te_ref, lv_ref, hm_ref, dw_ref, o_ref, dwb_ref):
    i = pl.program_id(0)
    changed = (i == 0) | (te_ref[i] != te_ref[jnp.maximum(i - 1, 0)])

    @pl.when(changed)
    def _cast_down():
        dwb_ref[...] = dw_ref[0].astype(jnp.bfloat16)

    @pl.when(lv_ref[i] > 0)
    def _compute():
        o_ref[...] = lax.dot_general(
            hm_ref[...], dwb_ref[...], (((1,), (1,)), ((), ())),
            preferred_element_type=jnp.float32,
        )


TPW = TOKENS // NW
NCH = 4
CHT = TPW // NCH


@functools.partial(
    pl.kernel,
    out_type=jax.ShapeDtypeStruct((TOKENS, HIDDEN), jnp.float32),
    mesh=_mesh,
    scratch_types=[
        pltpu.VMEM((NCH, 2 * CHT), jnp.int32),
        pltpu.VMEM((TPW, 128), jnp.float32),
        pltpu.VMEM((2 * CHT, HIDDEN), jnp.float32),
        pltpu.VMEM((2 * CHT, HIDDEN), jnp.float32),
        pltpu.VMEM((CHT, HIDDEN), jnp.float32),
        pltpu.SemaphoreType.DMA,
        pltpu.SemaphoreType.DMA,
    ],
)
def _combine(pos_hbm, w_hbm, y_hbm, out_hbm, idx_v, w_v, buf0_v, buf1_v, o_v,
             sem0, sem1):
    base = _wid() * TPW
    for ch in range(NCH):
        pltpu.sync_copy(pos_hbm.at[pl.ds(2 * base + ch * 2 * CHT, 2 * CHT)],
                        idx_v.at[ch])
    pltpu.sync_copy(w_hbm.at[pl.ds(base, TPW)], w_v)
    for ch in range(NCH):
        for c in range((2 * CHT) // LANES):
            v = idx_v[ch, pl.ds(c * LANES, LANES)]
            idx_v[ch, pl.ds(c * LANES, LANES)] = jnp.minimum(
                jnp.maximum(v, 0), PAD - 1)
    bufs = (buf0_v, buf1_v)
    sems = (sem0, sem1)
    cps = [pltpu.async_copy(y_hbm.at[idx_v.at[ch]], bufs[ch], sems[ch])
           for ch in range(2)]
    for ch in range(NCH):
        cps[ch].wait()
        buf_v = bufs[ch % 2]
        for j in range(CHT):
            p = ch * CHT + j
            w0 = w_v[p, pl.ds(0, LANES)]
            w1 = w_v[p, pl.ds(LANES, LANES)]

            def _col(cb, carry, j=j, w0=w0, w1=w1, buf_v=buf_v):
                s = cb * 64
                for u in range(4):
                    sl = pl.ds(s + u * LANES, LANES)
                    o_v[j, sl] = buf_v[2 * j, sl] * w0 + buf_v[2 * j + 1, sl] * w1
                return carry

            lax.fori_loop(0, HIDDEN // 64, _col, 0)
        pltpu.sync_copy(o_v, out_hbm.at[pl.ds(base + ch * CHT, CHT)])
        if ch + 2 < NCH:
            cps.append(pltpu.async_copy(
                y_hbm.at[idx_v.at[ch + 2]], bufs[ch % 2], sems[ch % 2]))


@jax.jit
def kernel(hidden_states, router, gate_up_proj, down_proj):
    pos_pad, w_pad, te_pad, lv_pad = pl.pallas_call(
        _route_sort_body,
        out_shape=[
            jax.ShapeDtypeStruct((TOKENS, 128), jnp.int32),
            jax.ShapeDtypeStruct((TOKENS, 128), jnp.float32),
            jax.ShapeDtypeStruct((128, 128), jnp.int32),
            jax.ShapeDtypeStruct((128, 128), jnp.int32),
        ],
    )(hidden_states, router)

    pos_flat = pos_pad[:, :TOP_K].reshape(NPAIR)
    te_arr = te_pad[:NT, 0]
    lv_arr = lv_pad[:NT, 0]

    hb = hidden_states.astype(jnp.bfloat16)
    hmid_sorted = pl.pallas_call(
        _mlp_up_body,
        grid_spec=pltpu.PrefetchScalarGridSpec(
            num_scalar_prefetch=2,
            grid=(NT,),
            in_specs=[
                pl.BlockSpec((TOKENS, 128), lambda i, te, lv: (0, 0)),
                pl.BlockSpec((TOKENS, HIDDEN), lambda i, te, lv: (0, 0)),
                pl.BlockSpec((1, 2 * INTER, HIDDEN),
                             lambda i, te, lv: (te[i], 0, 0)),
            ],
            out_specs=pl.BlockSpec((TILE, INTER), lambda i, te, lv: (i, 0)),
            scratch_shapes=[pltpu.VMEM((2 * INTER, HIDDEN), jnp.bfloat16)],
        ),
        out_shape=jax.ShapeDtypeStruct((PAD, INTER), jnp.bfloat16),
    )(te_arr, lv_arr, pos_pad, hb, gate_up_proj)

    y_sorted = pl.pallas_call(
        _mlp_down_body,
        grid_spec=pltpu.PrefetchScalarGridSpec(
            num_scalar_prefetch=2,
            grid=(NT,),
            in_specs=[
                pl.BlockSpec((TILE, INTER), lambda i, te, lv: (i, 0)),
                pl.BlockSpec((1, HIDDEN, INTER),
                             lambda i, te, lv: (te[i], 0, 0)),
            ],
            out_specs=pl.BlockSpec((TILE, HIDDEN), lambda i, te, lv: (i, 0)),
            scratch_shapes=[pltpu.VMEM((HIDDEN, INTER), jnp.bfloat16)],
        ),
        out_shape=jax.ShapeDtypeStruct((PAD, HIDDEN), jnp.float32),
    )(te_arr, lv_arr, hmid_sorted, down_proj)

    return _combine(pos_flat, w_pad, y_sorted)
